# Initial kernel scaffold; baseline (speedup 1.0000x reference)
import jax
import jax.numpy as jnp
from jax.experimental import pallas as pl


def kernel(u, v, n, U_emb, Q_emb, V_emb, T_emb, W, b):
    raise NotImplementedError("write your pallas kernel here")


# speedup vs baseline: 1.3744x; 1.3744x over previous
import functools

import jax
import jax.numpy as jnp
from jax import lax
from jax.experimental import pallas as pl
from jax.experimental.pallas import tpu as pltpu
from jax.experimental.pallas import tpu_sc as plsc

EMB = 128
NC = 2
NS = 16
NW = NC * NS


def _make_sc_gather(B, BK):
    bpw = B // NW
    npw = BK // NW
    CH = 128
    n_chunks = npw // CH

    mesh = plsc.VectorSubcoreMesh(core_axis_name="c", subcore_axis_name="s")

    @functools.partial(
        pl.kernel,
        mesh=mesh,
        out_type=[
            jax.ShapeDtypeStruct((B, EMB), jnp.float32),
            jax.ShapeDtypeStruct((B, EMB), jnp.float32),
            jax.ShapeDtypeStruct((B, EMB), jnp.float32),
            jax.ShapeDtypeStruct((B, EMB), jnp.float32),
            jax.ShapeDtypeStruct((BK, EMB), jnp.float32),
            jax.ShapeDtypeStruct((BK, EMB), jnp.float32),
        ],
        scratch_types=[
            pltpu.VMEM((bpw,), jnp.int32),
            pltpu.VMEM((n_chunks, CH), jnp.int32),
            pltpu.VMEM((CH, EMB), jnp.float32),
            pltpu.SemaphoreType.DMA,
        ],
    )
    def sc_gather(u1, v1, n3, Q_hbm, U_hbm, T_hbm, V_hbm,
                  q_out, ue_out, t_out, ve_out, tn_out, vn_out,
                  idx1, idxn, rows, sem):
        cid = lax.axis_index("c")
        sid = lax.axis_index("s")
        wid = sid * NC + cid
        base = wid * bpw

        pltpu.sync_copy(u1.at[pl.ds(base, bpw)], idx1)
        pltpu.async_copy(Q_hbm.at[idx1], rows, sem).wait()
        pltpu.sync_copy(rows, q_out.at[pl.ds(base, bpw)])
        pltpu.async_copy(U_hbm.at[idx1], rows, sem).wait()
        pltpu.sync_copy(rows, ue_out.at[pl.ds(base, bpw)])

        pltpu.sync_copy(v1.at[pl.ds(base, bpw)], idx1)
        pltpu.async_copy(T_hbm.at[idx1], rows, sem).wait()
        pltpu.sync_copy(rows, t_out.at[pl.ds(base, bpw)])
        pltpu.async_copy(V_hbm.at[idx1], rows, sem).wait()
        pltpu.sync_copy(rows, ve_out.at[pl.ds(base, bpw)])

        pltpu.sync_copy(n3.at[wid], idxn)
        nbase = wid * npw
        for j in range(n_chunks):
            pltpu.async_copy(T_hbm.at[idxn.at[j]], rows, sem).wait()
            pltpu.sync_copy(rows, tn_out.at[pl.ds(nbase + j * CH, CH)])
            pltpu.async_copy(V_hbm.at[idxn.at[j]], rows, sem).wait()
            pltpu.sync_copy(rows, vn_out.at[pl.ds(nbase + j * CH, CH)])

    return sc_gather


def _tc_body(K, q_ref, ue_ref, t_ref, ve_ref, vn_ref, tn_ref, w_ref, b_ref,
             pred_ref, predn_ref):
    q = q_ref[...]
    ue = ue_ref[...]
    w = w_ref[...]
    b = b_ref[...]

    h = ue * ve_ref[...]
    hw = lax.dot_general(h, w, (((1,), (1,)), ((), ())),
                         preferred_element_type=jnp.float32)
    s = (q * (hw + b)) * h
    pred_ref[...] = jnp.sum(s * jax.nn.sigmoid(t_ref[...]), axis=1,
                            keepdims=True)

    hn = jnp.repeat(ue, K, axis=0) * vn_ref[...]
    hnw = lax.dot_general(hn, w, (((1,), (1,)), ((), ())),
                          preferred_element_type=jnp.float32)
    sn = (jnp.repeat(q, K, axis=0) * (hnw + b)) * hn
    predn_ref[...] = jnp.sum(sn * jax.nn.sigmoid(tn_ref[...]), axis=1,
                             keepdims=True)


def _make_tc_compute(B, K, Bb=128):
    nb = B // Bb
    BKb = Bb * K

    return pl.pallas_call(
        functools.partial(_tc_body, K),
        grid=(nb,),
        in_specs=[
            pl.BlockSpec((Bb, EMB), lambda i: (i, 0)),
            pl.BlockSpec((Bb, EMB), lambda i: (i, 0)),
            pl.BlockSpec((Bb, EMB), lambda i: (i, 0)),
            pl.BlockSpec((Bb, EMB), lambda i: (i, 0)),
            pl.BlockSpec((BKb, EMB), lambda i: (i, 0)),
            pl.BlockSpec((BKb, EMB), lambda i: (i, 0)),
            pl.BlockSpec((EMB, EMB), lambda i: (0, 0)),
            pl.BlockSpec((1, EMB), lambda i: (0, 0)),
        ],
        out_specs=[
            pl.BlockSpec((Bb, 1), lambda i: (i, 0)),
            pl.BlockSpec((BKb, 1), lambda i: (i, 0)),
        ],
        out_shape=[
            jax.ShapeDtypeStruct((B, 1), jnp.float32),
            jax.ShapeDtypeStruct((B * K, 1), jnp.float32),
        ],
    )


def kernel(u, v, n, U_emb, Q_emb, V_emb, T_emb, W, b):
    B = u.shape[0]
    K = n.shape[1]
    BK = B * K

    u1 = u.astype(jnp.int32)
    v1 = v.astype(jnp.int32)
    n3 = n.astype(jnp.int32).reshape(NW, BK // NW // 128, 128)

    sc_gather = _make_sc_gather(B, BK)
    q, ue, t, ve, tn, vn = sc_gather(u1, v1, n3, Q_emb, U_emb, T_emb, V_emb)

    tc = _make_tc_compute(B, K)
    pred, predn = tc(q, ue, t, ve, vn, tn, W, b.reshape(1, EMB))
    return (pred.reshape(B), predn.reshape(BK))


# speedup vs baseline: 1.4318x; 1.0417x over previous
import functools

import jax
import jax.numpy as jnp
from jax import lax
from jax.experimental import pallas as pl
from jax.experimental.pallas import tpu as pltpu
from jax.experimental.pallas import tpu_sc as plsc

EMB = 128
NC = 2
NS = 16
NW = NC * NS
S = 4


def _make_sc_gather(Bs, BKs):
    bpw = Bs // NW
    npw = BKs // NW
    CH = 128
    n_chunks = npw // CH

    mesh = plsc.VectorSubcoreMesh(core_axis_name="c", subcore_axis_name="s")

    @functools.partial(
        pl.kernel,
        mesh=mesh,
        out_type=[
            jax.ShapeDtypeStruct((Bs, EMB), jnp.float32),
            jax.ShapeDtypeStruct((Bs, EMB), jnp.float32),
            jax.ShapeDtypeStruct((Bs, EMB), jnp.float32),
            jax.ShapeDtypeStruct((Bs, EMB), jnp.float32),
            jax.ShapeDtypeStruct((BKs, EMB), jnp.float32),
            jax.ShapeDtypeStruct((BKs, EMB), jnp.float32),
        ],
        scratch_types=[
            pltpu.VMEM((bpw,), jnp.int32),
            pltpu.VMEM((n_chunks, CH), jnp.int32),
            pltpu.VMEM((CH, EMB), jnp.float32),
            pltpu.SemaphoreType.DMA,
        ],
    )
    def sc_gather(u1, v1, n3, Q_hbm, U_hbm, T_hbm, V_hbm,
                  q_out, ue_out, t_out, ve_out, tn_out, vn_out,
                  idx1, idxn, rows, sem):
        cid = lax.axis_index("c")
        sid = lax.axis_index("s")
        wid = sid * NC + cid
        base = wid * bpw

        pltpu.sync_copy(u1.at[pl.ds(base, bpw)], idx1)
        pltpu.async_copy(Q_hbm.at[idx1], rows.at[pl.ds(0, bpw)], sem).wait()
        pltpu.sync_copy(rows.at[pl.ds(0, bpw)], q_out.at[pl.ds(base, bpw)])
        pltpu.async_copy(U_hbm.at[idx1], rows.at[pl.ds(0, bpw)], sem).wait()
        pltpu.sync_copy(rows.at[pl.ds(0, bpw)], ue_out.at[pl.ds(base, bpw)])

        pltpu.sync_copy(v1.at[pl.ds(base, bpw)], idx1)
        pltpu.async_copy(T_hbm.at[idx1], rows.at[pl.ds(0, bpw)], sem).wait()
        pltpu.sync_copy(rows.at[pl.ds(0, bpw)], t_out.at[pl.ds(base, bpw)])
        pltpu.async_copy(V_hbm.at[idx1], rows.at[pl.ds(0, bpw)], sem).wait()
        pltpu.sync_copy(rows.at[pl.ds(0, bpw)], ve_out.at[pl.ds(base, bpw)])

        pltpu.sync_copy(n3.at[wid], idxn)
        nbase = wid * npw
        for j in range(n_chunks):
            pltpu.async_copy(T_hbm.at[idxn.at[j]], rows, sem).wait()
            pltpu.sync_copy(rows, tn_out.at[pl.ds(nbase + j * CH, CH)])
            pltpu.async_copy(V_hbm.at[idxn.at[j]], rows, sem).wait()
            pltpu.sync_copy(rows, vn_out.at[pl.ds(nbase + j * CH, CH)])

    return sc_gather


def _tc_body(K, Bb, q_ref, ue_ref, t_ref, ve_ref, vn_ref, tn_ref, w_ref, b_ref,
             pred_ref, predn_ref):
    q = q_ref[...]
    ue = ue_ref[...]
    w = w_ref[...]
    b = b_ref[...]

    h = ue * ve_ref[...]
    hw = lax.dot_general(h, w, (((1,), (1,)), ((), ())),
                         preferred_element_type=jnp.float32)
    s = (q * (hw + b)) * h
    pred_ref[...] = jnp.sum(s * jax.nn.sigmoid(t_ref[...]), axis=1,
                            keepdims=True)

    hn = vn_ref[...] * ue[None]
    hnw = lax.dot_general(hn.reshape(K * Bb, EMB), w, (((1,), (1,)), ((), ())),
                          preferred_element_type=jnp.float32)
    an = (hnw.reshape(K, Bb, EMB) + b[None]) * q[None]
    sn = an * hn
    predn_ref[...] = jnp.sum(sn * jax.nn.sigmoid(tn_ref[...]), axis=2,
                             keepdims=True)


def _make_tc_compute(Bs, K, Bb=128):
    nb = Bs // Bb

    return pl.pallas_call(
        functools.partial(_tc_body, K, Bb),
        grid=(nb,),
        in_specs=[
            pl.BlockSpec((Bb, EMB), lambda i: (i, 0)),
            pl.BlockSpec((Bb, EMB), lambda i: (i, 0)),
            pl.BlockSpec((Bb, EMB), lambda i: (i, 0)),
            pl.BlockSpec((Bb, EMB), lambda i: (i, 0)),
            pl.BlockSpec((K, Bb, EMB), lambda i: (0, i, 0)),
            pl.BlockSpec((K, Bb, EMB), lambda i: (0, i, 0)),
            pl.BlockSpec((EMB, EMB), lambda i: (0, 0)),
            pl.BlockSpec((1, EMB), lambda i: (0, 0)),
        ],
        out_specs=[
            pl.BlockSpec((Bb, 1), lambda i: (i, 0)),
            pl.BlockSpec((K, Bb, 1), lambda i: (0, i, 0)),
        ],
        out_shape=[
            jax.ShapeDtypeStruct((Bs, 1), jnp.float32),
            jax.ShapeDtypeStruct((K, Bs, 1), jnp.float32),
        ],
    )


def kernel(u, v, n, U_emb, Q_emb, V_emb, T_emb, W, b):
    B = u.shape[0]
    K = n.shape[1]
    Bs = B // S
    BKs = Bs * K

    u1 = u.astype(jnp.int32)
    v1 = v.astype(jnp.int32)
    n32 = n.astype(jnp.int32)
    b2 = b.reshape(1, EMB)

    sc_gather = _make_sc_gather(Bs, BKs)
    tc = _make_tc_compute(Bs, K)

    preds = []
    predns = []
    for s in range(S):
        sl = slice(s * Bs, (s + 1) * Bs)
        nT = n32[sl].T.reshape(NW, BKs // NW // 128, 128)
        q, ue, t, ve, tn, vn = sc_gather(u1[sl], v1[sl], nT,
                                         Q_emb, U_emb, T_emb, V_emb)
        pred_s, prednT_s = tc(q, ue, t, ve,
                              vn.reshape(K, Bs, EMB), tn.reshape(K, Bs, EMB),
                              W, b2)
        preds.append(pred_s.reshape(Bs))
        predns.append(prednT_s.reshape(K, Bs).T)

    pred = jnp.concatenate(preds)
    predn = jnp.concatenate(predns).reshape(B * K)
    return (pred, predn)


# speedup vs baseline: 1.5909x; 1.1111x over previous
import functools

import jax
import jax.numpy as jnp
from jax import lax
from jax.experimental import pallas as pl
from jax.experimental.pallas import tpu as pltpu
from jax.experimental.pallas import tpu_sc as plsc

EMB = 128
NC = 2
NS = 16
NW = NC * NS
S = 2


def _make_sc_gather(Bs, BKs):
    bpw = Bs // NW
    npw = BKs // NW
    CH = 128
    n_chunks = npw // CH

    mesh = plsc.VectorSubcoreMesh(core_axis_name="c", subcore_axis_name="s")

    @functools.partial(
        pl.kernel,
        mesh=mesh,
        out_type=[
            jax.ShapeDtypeStruct((Bs, EMB), jnp.float32),
            jax.ShapeDtypeStruct((Bs, EMB), jnp.float32),
            jax.ShapeDtypeStruct((Bs, EMB), jnp.float32),
            jax.ShapeDtypeStruct((Bs, EMB), jnp.float32),
            jax.ShapeDtypeStruct((BKs, EMB), jnp.float32),
            jax.ShapeDtypeStruct((BKs, EMB), jnp.float32),
        ],
        scratch_types=[
            pltpu.VMEM((bpw,), jnp.int32),
            pltpu.VMEM((bpw,), jnp.int32),
            pltpu.VMEM((n_chunks, CH), jnp.int32),
            pltpu.VMEM((CH, EMB), jnp.float32),
            pltpu.VMEM((CH, EMB), jnp.float32),
            pltpu.SemaphoreType.DMA,
            pltpu.SemaphoreType.DMA,
            pltpu.SemaphoreType.DMA,
        ],
    )
    def sc_gather(u1, v1, n3, Q_hbm, U_hbm, T_hbm, V_hbm,
                  q_out, ue_out, t_out, ve_out, tn_out, vn_out,
                  idxu, idxv, idxn, rowsA, rowsB, gsem, wsemA, wsemB):
        cid = lax.axis_index("c")
        sid = lax.axis_index("s")
        wid = sid * NC + cid
        base = wid * bpw
        nbase = wid * npw

        pltpu.sync_copy(u1.at[pl.ds(base, bpw)], idxu)
        pltpu.sync_copy(v1.at[pl.ds(base, bpw)], idxv)
        pltpu.sync_copy(n3.at[wid], idxn)

        tasks = [
            (Q_hbm, idxu, q_out, base, bpw),
            (U_hbm, idxu, ue_out, base, bpw),
            (T_hbm, idxv, t_out, base, bpw),
            (V_hbm, idxv, ve_out, base, bpw),
        ]
        for j in range(n_chunks):
            tasks.append((T_hbm, idxn.at[j], tn_out, nbase + j * CH, CH))
            tasks.append((V_hbm, idxn.at[j], vn_out, nbase + j * CH, CH))

        bufs = (rowsA, rowsB)
        wsems = (wsemA, wsemB)
        pending = [None, None]
        for j, (tbl, idx, out, off, ln) in enumerate(tasks):
            p = j % 2
            buf = bufs[p] if ln == CH else bufs[p].at[pl.ds(0, ln)]
            if pending[p] is not None:
                pending[p].wait()
            pltpu.async_copy(tbl.at[idx], buf, gsem).wait()
            pending[p] = pltpu.async_copy(buf, out.at[pl.ds(off, ln)],
                                          wsems[p])
        for p in (0, 1):
            if pending[p] is not None:
                pending[p].wait()

    return sc_gather


def _tc_body(K, Bb, q_ref, ue_ref, t_ref, ve_ref, vn_ref, tn_ref, w_ref, b_ref,
             pred_ref, predn_ref):
    q = q_ref[...]
    ue = ue_ref[...]
    w = w_ref[...]
    b = b_ref[...]

    h = ue * ve_ref[...]
    hw = lax.dot_general(h, w, (((1,), (1,)), ((), ())),
                         preferred_element_type=jnp.float32)
    s = (q * (hw + b)) * h
    pred_ref[...] = jnp.sum(s * jax.nn.sigmoid(t_ref[...]), axis=1,
                            keepdims=True)

    hn = vn_ref[...] * ue[None]
    hnw = lax.dot_general(hn.reshape(K * Bb, EMB), w, (((1,), (1,)), ((), ())),
                          preferred_element_type=jnp.float32)
    an = (hnw.reshape(K, Bb, EMB) + b[None]) * q[None]
    sn = an * hn
    predn_ref[...] = jnp.sum(sn * jax.nn.sigmoid(tn_ref[...]), axis=2,
                             keepdims=True)


def _make_tc_compute(Bs, K, Bb=128):
    nb = Bs // Bb

    return pl.pallas_call(
        functools.partial(_tc_body, K, Bb),
        grid=(nb,),
        in_specs=[
            pl.BlockSpec((Bb, EMB), lambda i: (i, 0)),
            pl.BlockSpec((Bb, EMB), lambda i: (i, 0)),
            pl.BlockSpec((Bb, EMB), lambda i: (i, 0)),
            pl.BlockSpec((Bb, EMB), lambda i: (i, 0)),
            pl.BlockSpec((K, Bb, EMB), lambda i: (0, i, 0)),
            pl.BlockSpec((K, Bb, EMB), lambda i: (0, i, 0)),
            pl.BlockSpec((EMB, EMB), lambda i: (0, 0)),
            pl.BlockSpec((1, EMB), lambda i: (0, 0)),
        ],
        out_specs=[
            pl.BlockSpec((Bb, 1), lambda i: (i, 0)),
            pl.BlockSpec((K, Bb, 1), lambda i: (0, i, 0)),
        ],
        out_shape=[
            jax.ShapeDtypeStruct((Bs, 1), jnp.float32),
            jax.ShapeDtypeStruct((K, Bs, 1), jnp.float32),
        ],
    )


def kernel(u, v, n, U_emb, Q_emb, V_emb, T_emb, W, b):
    B = u.shape[0]
    K = n.shape[1]
    Bs = B // S
    BKs = Bs * K

    u1 = u.astype(jnp.int32)
    v1 = v.astype(jnp.int32)
    n32 = n.astype(jnp.int32)
    b2 = b.reshape(1, EMB)

    sc_gather = _make_sc_gather(Bs, BKs)
    tc = _make_tc_compute(Bs, K)

    preds = []
    predns = []
    for s in range(S):
        sl = slice(s * Bs, (s + 1) * Bs)
        nT = n32[sl].T.reshape(NW, BKs // NW // 128, 128)
        q, ue, t, ve, tn, vn = sc_gather(u1[sl], v1[sl], nT,
                                         Q_emb, U_emb, T_emb, V_emb)
        pred_s, prednT_s = tc(q, ue, t, ve,
                              vn.reshape(K, Bs, EMB), tn.reshape(K, Bs, EMB),
                              W, b2)
        preds.append(pred_s.reshape(Bs))
        predns.append(prednT_s.reshape(K, Bs).T)

    pred = jnp.concatenate(preds)
    predn = jnp.concatenate(predns).reshape(B * K)
    return (pred, predn)


# speedup vs baseline: 1.6971x; 1.0668x over previous
import functools

import jax
import jax.numpy as jnp
from jax import lax
from jax.experimental import pallas as pl
from jax.experimental.pallas import tpu as pltpu
from jax.experimental.pallas import tpu_sc as plsc

EMB = 128
NC = 2
NS = 16
NW = NC * NS
S = 2


def _make_sc_gather(Bs, BKs):
    bpw = Bs // NW
    npw = BKs // NW
    CH = 128
    n_chunks = npw // CH

    mesh = plsc.VectorSubcoreMesh(core_axis_name="c", subcore_axis_name="s")

    NBUF = 4
    LAG = 2

    @functools.partial(
        pl.kernel,
        mesh=mesh,
        out_type=[
            jax.ShapeDtypeStruct((Bs, EMB), jnp.float32),
            jax.ShapeDtypeStruct((Bs, EMB), jnp.float32),
            jax.ShapeDtypeStruct((Bs, EMB), jnp.float32),
            jax.ShapeDtypeStruct((Bs, EMB), jnp.float32),
            jax.ShapeDtypeStruct((BKs, EMB), jnp.float32),
            jax.ShapeDtypeStruct((BKs, EMB), jnp.float32),
        ],
        scratch_types=(
            [pltpu.VMEM((bpw,), jnp.int32),
             pltpu.VMEM((bpw,), jnp.int32),
             pltpu.VMEM((n_chunks, CH), jnp.int32)]
            + [pltpu.VMEM((CH, EMB), jnp.float32) for _ in range(NBUF)]
            + [pltpu.SemaphoreType.DMA for _ in range(2 * NBUF)]
        ),
    )
    def sc_gather(u1, v1, n3, Q_hbm, U_hbm, T_hbm, V_hbm,
                  q_out, ue_out, t_out, ve_out, tn_out, vn_out,
                  idxu, idxv, idxn, *bufs_and_sems):
        bufs = bufs_and_sems[:NBUF]
        gsems = bufs_and_sems[NBUF:2 * NBUF]
        wsems = bufs_and_sems[2 * NBUF:]
        cid = lax.axis_index("c")
        sid = lax.axis_index("s")
        wid = sid * NC + cid
        base = wid * bpw
        nbase = wid * npw

        pltpu.sync_copy(u1.at[pl.ds(base, bpw)], idxu)
        pltpu.sync_copy(v1.at[pl.ds(base, bpw)], idxv)
        pltpu.sync_copy(n3.at[wid], idxn)

        tasks = [
            (Q_hbm, idxu, q_out, base, bpw),
            (U_hbm, idxu, ue_out, base, bpw),
            (T_hbm, idxv, t_out, base, bpw),
            (V_hbm, idxv, ve_out, base, bpw),
        ]
        for j in range(n_chunks):
            tasks.append((T_hbm, idxn.at[j], tn_out, nbase + j * CH, CH))
            tasks.append((V_hbm, idxn.at[j], vn_out, nbase + j * CH, CH))

        gpend = [None] * NBUF
        wpend = [None] * NBUF
        views = [None] * NBUF

        def drain(j):
            p = j % NBUF
            gpend[p].wait()
            buf, out, off, ln = views[p]
            wpend[p] = pltpu.async_copy(buf, out.at[pl.ds(off, ln)], wsems[p])

        for j, (tbl, idx, out, off, ln) in enumerate(tasks):
            p = j % NBUF
            buf = bufs[p] if ln == CH else bufs[p].at[pl.ds(0, ln)]
            if wpend[p] is not None:
                wpend[p].wait()
            gpend[p] = pltpu.async_copy(tbl.at[idx], buf, gsems[p])
            views[p] = (buf, out, off, ln)
            if j >= LAG:
                drain(j - LAG)
        for j in range(len(tasks) - LAG, len(tasks)):
            drain(j)
        for p in range(NBUF):
            if wpend[p] is not None:
                wpend[p].wait()

    return sc_gather


def _tc_body(K, Bb, q_ref, ue_ref, t_ref, ve_ref, vn_ref, tn_ref, w_ref, b_ref,
             pred_ref, predn_ref):
    q = q_ref[...]
    ue = ue_ref[...]
    w = w_ref[...]
    b = b_ref[...]

    h = ue * ve_ref[...]
    hw = lax.dot_general(h, w, (((1,), (1,)), ((), ())),
                         preferred_element_type=jnp.float32)
    s = (q * (hw + b)) * h
    pred_ref[...] = jnp.sum(s * jax.nn.sigmoid(t_ref[...]), axis=1,
                            keepdims=True)

    hn = vn_ref[...] * ue[None]
    hnw = lax.dot_general(hn.reshape(K * Bb, EMB), w, (((1,), (1,)), ((), ())),
                          preferred_element_type=jnp.float32)
    an = (hnw.reshape(K, Bb, EMB) + b[None]) * q[None]
    sn = an * hn
    predn_ref[...] = jnp.sum(sn * jax.nn.sigmoid(tn_ref[...]), axis=2,
                             keepdims=True)


def _make_tc_compute(Bs, K, Bb=128):
    nb = Bs // Bb

    return pl.pallas_call(
        functools.partial(_tc_body, K, Bb),
        grid=(nb,),
        in_specs=[
            pl.BlockSpec((Bb, EMB), lambda i: (i, 0)),
            pl.BlockSpec((Bb, EMB), lambda i: (i, 0)),
            pl.BlockSpec((Bb, EMB), lambda i: (i, 0)),
            pl.BlockSpec((Bb, EMB), lambda i: (i, 0)),
            pl.BlockSpec((K, Bb, EMB), lambda i: (0, i, 0)),
            pl.BlockSpec((K, Bb, EMB), lambda i: (0, i, 0)),
            pl.BlockSpec((EMB, EMB), lambda i: (0, 0)),
            pl.BlockSpec((1, EMB), lambda i: (0, 0)),
        ],
        out_specs=[
            pl.BlockSpec((Bb, 1), lambda i: (i, 0)),
            pl.BlockSpec((K, Bb, 1), lambda i: (0, i, 0)),
        ],
        out_shape=[
            jax.ShapeDtypeStruct((Bs, 1), jnp.float32),
            jax.ShapeDtypeStruct((K, Bs, 1), jnp.float32),
        ],
    )


def kernel(u, v, n, U_emb, Q_emb, V_emb, T_emb, W, b):
    B = u.shape[0]
    K = n.shape[1]
    Bs = B // S
    BKs = Bs * K

    u1 = u.astype(jnp.int32)
    v1 = v.astype(jnp.int32)
    n32 = n.astype(jnp.int32)
    b2 = b.reshape(1, EMB)

    sc_gather = _make_sc_gather(Bs, BKs)
    tc = _make_tc_compute(Bs, K)

    preds = []
    predns = []
    for s in range(S):
        sl = slice(s * Bs, (s + 1) * Bs)
        nT = n32[sl].T.reshape(NW, BKs // NW // 128, 128)
        q, ue, t, ve, tn, vn = sc_gather(u1[sl], v1[sl], nT,
                                         Q_emb, U_emb, T_emb, V_emb)
        pred_s, prednT_s = tc(q, ue, t, ve,
                              vn.reshape(K, Bs, EMB), tn.reshape(K, Bs, EMB),
                              W, b2)
        preds.append(pred_s.reshape(Bs))
        predns.append(prednT_s.reshape(K, Bs).T)

    pred = jnp.concatenate(preds)
    predn = jnp.concatenate(predns).reshape(B * K)
    return (pred, predn)
